# baseline (speedup 1.0000x reference)
import functools

import jax
import jax.numpy as jnp
from jax import lax
from jax.experimental import pallas as pl
from jax.experimental.pallas import tpu as pltpu
from jax.experimental.pallas import tpu_sc as plsc

N = 10000
E = 320000
D = 128
H = 8
OUT = 16
ROW = 136

NC = 2
NS = 16
NW = NC * NS
EPW = E // NW
B = 80
B2 = B // 2
NCHUNK = EPW // B
EROW = 2 * B
IBLK1 = (NCHUNK + 1) // 2
NZC = N // B
HALF = N // NC
FCH = B2
NFC = HALF // FCH

_LANES = 16


def _lane_bcast(v, lane):
  return jnp.broadcast_to(v[lane], (_LANES,))


def _prep_body(h_ref, w_ref, pl_ref, pr_ref, featel_ref, er_ref):
  feat = jnp.dot(h_ref[...], w_ref[...], preferred_element_type=jnp.float32)
  el8 = jnp.dot(feat, pl_ref[...], preferred_element_type=jnp.float32)
  er8 = jnp.dot(feat, pr_ref[...], preferred_element_type=jnp.float32)
  featel_ref[...] = jnp.concatenate([feat, el8], axis=1)
  er_ref[...] = jnp.concatenate([jnp.zeros_like(er8), er8], axis=1)


_PREP_BLK = 1000

_prep = pl.pallas_call(
    _prep_body,
    grid=(N // _PREP_BLK,),
    in_specs=[
        pl.BlockSpec((_PREP_BLK, D), lambda i: (i, 0)),
        pl.BlockSpec((D, D), lambda i: (0, 0)),
        pl.BlockSpec((D, H), lambda i: (0, 0)),
        pl.BlockSpec((D, H), lambda i: (0, 0)),
    ],
    out_specs=[
        pl.BlockSpec((_PREP_BLK, ROW), lambda i: (i, 0)),
        pl.BlockSpec((_PREP_BLK, 16), lambda i: (i, 0)),
    ],
    out_shape=[
        jax.ShapeDtypeStruct((N, ROW), jnp.float32),
        jax.ShapeDtypeStruct((N, 16), jnp.float32),
    ],
)


def _sc_body(featel_hbm, er_hbm, src_hbm, dst_hbm, bias_hbm,
             out_hbm, xchg_hbm,
             acc, sidxb, didxb, g2, r2, o_buf, pidx, semg, semr, sems,
             semb):
  cid = lax.axis_index("c")
  sid = lax.axis_index("s")
  wid = cid * NS + sid

  zv = jnp.zeros((_LANES,), jnp.float32)

  def _zero_row(i, _):
    for q in range(2):
      for c in range(H):
        o_buf[q, i, pl.ds(c * _LANES, _LANES)] = zv
      o_buf[q, i, pl.ds(ROW - _LANES, _LANES)] = zv
    return _

  lax.fori_loop(0, B2, _zero_row, None)

  def _zero_chunk(j, _):
    c = sid + j * NS

    @pl.when(c < NZC)
    def _():
      pltpu.sync_copy(o_buf.at[0], acc.at[pl.ds(c * B, B2)])
      pltpu.sync_copy(o_buf.at[1], acc.at[pl.ds(c * B + B2, B2)])
    return _

  lax.fori_loop(0, pl.cdiv(NZC, NS), _zero_chunk, None)
  plsc.subcore_barrier()

  ziv = jnp.zeros((_LANES,), jnp.int32)
  pidx[pl.ds(0, _LANES)] = ziv
  pidx[pl.ds(_LANES, _LANES)] = ziv
  pidx[pl.ds(B2 - _LANES, _LANES)] = ziv
  for q in range(2):
    pltpu.async_copy(o_buf.at[q], acc.at[pidx], sems.at[q], add=True)

  def _issue_gather(bc, p):
    pltpu.async_copy(featel_hbm.at[sidxb.at[bc]], g2.at[p], semg.at[p])
    pltpu.async_copy(er_hbm.at[didxb.at[bc]], r2.at[p], semr.at[p])

  def _wait_gather(p):
    pltpu.make_async_copy(featel_hbm.at[sidxb.at[0]],
                          g2.at[p], semg.at[p]).wait()
    pltpu.make_async_copy(er_hbm.at[didxb.at[0]],
                          r2.at[p], semr.at[p]).wait()

  def _wait_scatter(q):
    pltpu.make_async_copy(o_buf.at[q], acc.at[pidx], sems.at[q]).wait()

  def _half(bc, p, q):
    _wait_scatter(q)

    def _edge(i, _):
      el = g2[p, q * B2 + i, pl.ds(ROW - _LANES, _LANES)]
      er = r2[p, q * B2 + i, pl.ds(0, _LANES)]
      sv = el + er
      sv = jnp.where(sv >= 0.0, sv, sv * jnp.float32(0.2))
      w = jnp.exp(sv)
      o_buf[q, i, pl.ds(ROW - _LANES, _LANES)] = w
      for hh in range(H):
        fh = g2[p, q * B2 + i, pl.ds(hh * OUT, _LANES)]
        o_buf[q, i, pl.ds(hh * OUT, _LANES)] = fh * _lane_bcast(w, 8 + hh)
      return _

    lax.fori_loop(0, B2, _edge, None)
    pltpu.async_copy(o_buf.at[q],
                     acc.at[didxb.at[bc, pl.ds(q * B2, B2)]],
                     sems.at[q], add=True)

  def _do_chunk(bc, p):
    _wait_gather(p)
    _half(bc, p, 0)
    _half(bc, p, 1)

  def _run_block(c_base, n):
    pltpu.sync_copy(src_hbm.at[pl.ds(wid * NCHUNK + c_base, n)],
                    sidxb.at[pl.ds(0, n)])
    pltpu.sync_copy(dst_hbm.at[pl.ds(wid * NCHUNK + c_base, n)],
                    didxb.at[pl.ds(0, n)])
    _issue_gather(jnp.int32(0), 0)
    npairs = (n - 1) // 2

    def _pair(jj, _):
      bc0 = 2 * jj
      bc1 = bc0 + 1
      _issue_gather(bc1, 1)
      _wait_gather(0)
      _half(bc0, 0, 0)
      _half(bc0, 0, 1)

      @pl.when(bc0 + 2 < n)
      def _():
        _issue_gather(bc0 + 2, 0)

      _wait_gather(1)
      _half(bc1, 1, 0)
      _half(bc1, 1, 1)
      return _

    lax.fori_loop(0, npairs, _pair, None)
    for bc in range(2 * npairs, n):
      p = bc % 2
      if bc > 2 * npairs:
        _issue_gather(jnp.int32(bc), p)
      _wait_gather(p)
      _half(jnp.int32(bc), p, 0)
      _half(jnp.int32(bc), p, 1)

  _run_block(0, IBLK1)
  _run_block(IBLK1, NCHUNK - IBLK1)
  _wait_scatter(0)
  _wait_scatter(1)
  plsc.subcore_barrier()

  other_base = (1 - cid) * HALF

  def _xchg_chunk(j, _):
    ch = sid + j * NS

    @pl.when(ch < NFC)
    def _():
      pltpu.sync_copy(acc.at[pl.ds(other_base + ch * FCH, FCH)],
                      xchg_hbm.at[cid, pl.ds(ch * FCH, FCH)])
    return _

  lax.fori_loop(0, pl.cdiv(NFC, NS), _xchg_chunk, None)
  plsc.subcore_barrier()
  pltpu.core_barrier(semb, core_axis_name="c")

  pltpu.sync_copy(bias_hbm, r2.at[0, pl.ds(0, H)])
  my_base = cid * HALF

  def _fin_chunk(j, _):
    ch = sid + j * NS

    @pl.when(ch < NFC)
    def _():
      a_buf = g2.at[0, pl.ds(0, FCH)]
      t_buf = g2.at[1, pl.ds(0, FCH)]
      pltpu.sync_copy(acc.at[pl.ds(my_base + ch * FCH, FCH)], a_buf)
      pltpu.sync_copy(xchg_hbm.at[1 - cid, pl.ds(ch * FCH, FCH)], t_buf)

      def _row(r, _):
        dn = (g2[0, r, pl.ds(ROW - _LANES, _LANES)]
              + g2[1, r, pl.ds(ROW - _LANES, _LANES)])
        for hh in range(H):
          sg = (g2[0, r, pl.ds(hh * OUT, _LANES)]
                + g2[1, r, pl.ds(hh * OUT, _LANES)])
          den = _lane_bcast(dn, 8 + hh)
          x = sg / jnp.maximum(den, jnp.float32(1e-38))
          x = x + r2[0, hh, pl.ds(0, _LANES)]
          o_buf[0, r, pl.ds(hh * OUT, _LANES)] = jnp.where(
              x > 0.0, x, jnp.exp(x) - jnp.float32(1.0))
        return _

      lax.fori_loop(0, FCH, _row, None)
      pltpu.sync_copy(o_buf.at[0],
                      out_hbm.at[pl.ds(my_base + ch * FCH, FCH)])
    return _

  lax.fori_loop(0, pl.cdiv(NFC, NS), _fin_chunk, None)


@functools.cache
def _make_sc_edge():
  return pl.kernel(
      _sc_body,
      out_type=[
          jax.ShapeDtypeStruct((N, ROW), jnp.float32),
          jax.ShapeDtypeStruct((NC, HALF, ROW), jnp.float32),
      ],
      mesh=plsc.VectorSubcoreMesh(
          core_axis_name="c", subcore_axis_name="s",
          num_cores=NC, num_subcores=NS),
      scratch_types=[
          pltpu.VMEM_SHARED((N, ROW), jnp.float32),
          pltpu.VMEM((IBLK1, B), jnp.int32),
          pltpu.VMEM((IBLK1, B), jnp.int32),
          pltpu.VMEM((2, B, ROW), jnp.float32),
          pltpu.VMEM((2, B, 16), jnp.float32),
          pltpu.VMEM((2, B2, ROW), jnp.float32),
          pltpu.VMEM((B2,), jnp.int32),
          pltpu.SemaphoreType.DMA((2,)),
          pltpu.SemaphoreType.DMA((2,)),
          pltpu.SemaphoreType.DMA((2,)),
          pltpu.SemaphoreType.REGULAR,
      ],
      compiler_params=pltpu.CompilerParams(use_tc_tiling_on_sc=False),
  )


def kernel(h, edge_index, W, attn_l, attn_r, bias):
  src = edge_index[0].astype(jnp.int32).reshape(E // B, B)
  dst = edge_index[1].astype(jnp.int32).reshape(E // B, B)

  mask = (jnp.arange(D)[:, None] // OUT == jnp.arange(H)[None, :])
  p_l = jnp.where(mask, attn_l.reshape(D, 1), 0.0)
  p_r = jnp.where(mask, attn_r.reshape(D, 1), 0.0)

  featel, er16 = _prep(h, W, p_l, p_r)
  out136, _unused = _make_sc_edge()(featel, er16, src, dst,
                                    bias.reshape(H, OUT))
  return out136[:, :D]


# speedup vs baseline: 1.0348x; 1.0348x over previous
import functools

import jax
import jax.numpy as jnp
from jax import lax
from jax.experimental import pallas as pl
from jax.experimental.pallas import tpu as pltpu
from jax.experimental.pallas import tpu_sc as plsc

N = 10000
E = 320000
D = 128
H = 8
OUT = 16
ROW = 136

NC = 2
NS = 16
NW = NC * NS
EPW = E // NW
B = 80
B2 = B // 2
NCHUNK = EPW // B
EROW = 2 * B
IBLK1 = (NCHUNK + 1) // 2
NZC = N // B

_LANES = 16


def _lane_bcast(v, lane):
  return jnp.broadcast_to(v[lane], (_LANES,))


def _prep_body(h_ref, w_ref, pl_ref, pr_ref, featel_ref, er_ref):
  feat = jnp.dot(h_ref[...], w_ref[...], preferred_element_type=jnp.float32)
  el8 = jnp.dot(feat, pl_ref[...], preferred_element_type=jnp.float32)
  er8 = jnp.dot(feat, pr_ref[...], preferred_element_type=jnp.float32)
  featel_ref[...] = jnp.concatenate([feat, el8], axis=1)
  er_ref[...] = jnp.concatenate([jnp.zeros_like(er8), er8], axis=1)


_PREP_BLK = 1000

_prep = pl.pallas_call(
    _prep_body,
    grid=(N // _PREP_BLK,),
    in_specs=[
        pl.BlockSpec((_PREP_BLK, D), lambda i: (i, 0)),
        pl.BlockSpec((D, D), lambda i: (0, 0)),
        pl.BlockSpec((D, H), lambda i: (0, 0)),
        pl.BlockSpec((D, H), lambda i: (0, 0)),
    ],
    out_specs=[
        pl.BlockSpec((_PREP_BLK, ROW), lambda i: (i, 0)),
        pl.BlockSpec((_PREP_BLK, 16), lambda i: (i, 0)),
    ],
    out_shape=[
        jax.ShapeDtypeStruct((N, ROW), jnp.float32),
        jax.ShapeDtypeStruct((N, 16), jnp.float32),
    ],
)


def _sc_body(featel_hbm, er_hbm, src_hbm, dst_hbm, out_hbm,
             acc, sidxb, didxb, g2, r2, o_buf, pidx, semg, semr, sems):
  cid = lax.axis_index("c")
  sid = lax.axis_index("s")
  wid = cid * NS + sid

  zv = jnp.zeros((_LANES,), jnp.float32)

  def _zero_row(i, _):
    for q in range(2):
      for c in range(H):
        o_buf[q, i, pl.ds(c * _LANES, _LANES)] = zv
      o_buf[q, i, pl.ds(ROW - _LANES, _LANES)] = zv
    return _

  lax.fori_loop(0, B2, _zero_row, None)

  def _zero_chunk(j, _):
    c = sid + j * NS

    @pl.when(c < NZC)
    def _():
      pltpu.sync_copy(o_buf.at[0], acc.at[pl.ds(c * B, B2)])
      pltpu.sync_copy(o_buf.at[1], acc.at[pl.ds(c * B + B2, B2)])
    return _

  lax.fori_loop(0, pl.cdiv(NZC, NS), _zero_chunk, None)
  plsc.subcore_barrier()

  ziv = jnp.zeros((_LANES,), jnp.int32)
  pidx[pl.ds(0, _LANES)] = ziv
  pidx[pl.ds(_LANES, _LANES)] = ziv
  pidx[pl.ds(B2 - _LANES, _LANES)] = ziv
  for q in range(2):
    pltpu.async_copy(o_buf.at[q], acc.at[pidx], sems.at[q], add=True)

  def _issue_gather(bc, p):
    pltpu.async_copy(featel_hbm.at[sidxb.at[bc]], g2.at[p], semg.at[p])
    pltpu.async_copy(er_hbm.at[didxb.at[bc]], r2.at[p], semr.at[p])

  def _wait_gather(p):
    pltpu.make_async_copy(featel_hbm.at[sidxb.at[0]],
                          g2.at[p], semg.at[p]).wait()
    pltpu.make_async_copy(er_hbm.at[didxb.at[0]],
                          r2.at[p], semr.at[p]).wait()

  def _wait_scatter(q):
    pltpu.make_async_copy(o_buf.at[q], acc.at[pidx], sems.at[q]).wait()

  def _half(bc, p, q):
    _wait_scatter(q)

    def _edge(i, _):
      el = g2[p, q * B2 + i, pl.ds(ROW - _LANES, _LANES)]
      er = r2[p, q * B2 + i, pl.ds(0, _LANES)]
      sv = el + er
      sv = jnp.where(sv >= 0.0, sv, sv * jnp.float32(0.2))
      w = jnp.exp(sv)
      o_buf[q, i, pl.ds(ROW - _LANES, _LANES)] = w
      for hh in range(H):
        fh = g2[p, q * B2 + i, pl.ds(hh * OUT, _LANES)]
        o_buf[q, i, pl.ds(hh * OUT, _LANES)] = fh * _lane_bcast(w, 8 + hh)
      return _

    lax.fori_loop(0, B2, _edge, None)
    pltpu.async_copy(o_buf.at[q],
                     acc.at[didxb.at[bc, pl.ds(q * B2, B2)]],
                     sems.at[q], add=True)

  def _do_chunk(bc, p):
    _wait_gather(p)
    _half(bc, p, 0)
    _half(bc, p, 1)

  def _run_block(c_base, n):
    pltpu.sync_copy(src_hbm.at[pl.ds(wid * NCHUNK + c_base, n)],
                    sidxb.at[pl.ds(0, n)])
    pltpu.sync_copy(dst_hbm.at[pl.ds(wid * NCHUNK + c_base, n)],
                    didxb.at[pl.ds(0, n)])
    _issue_gather(jnp.int32(0), 0)
    npairs = (n - 1) // 2

    def _pair(jj, _):
      bc0 = 2 * jj
      bc1 = bc0 + 1
      _issue_gather(bc1, 1)
      _wait_gather(0)
      _half(bc0, 0, 0)
      _half(bc0, 0, 1)

      @pl.when(bc0 + 2 < n)
      def _():
        _issue_gather(bc0 + 2, 0)

      _wait_gather(1)
      _half(bc1, 1, 0)
      _half(bc1, 1, 1)
      return _

    lax.fori_loop(0, npairs, _pair, None)
    for bc in range(2 * npairs, n):
      p = bc % 2
      if bc > 2 * npairs:
        _issue_gather(jnp.int32(bc), p)
      _wait_gather(p)
      _half(jnp.int32(bc), p, 0)
      _half(jnp.int32(bc), p, 1)

  _run_block(0, IBLK1)
  _run_block(IBLK1, NCHUNK - IBLK1)
  _wait_scatter(0)
  _wait_scatter(1)
  plsc.subcore_barrier()

  def _out_chunk(j, _):
    c = sid + j * NS

    @pl.when(c < NZC)
    def _():
      pltpu.sync_copy(acc.at[pl.ds(c * B, B)],
                      out_hbm.at[cid, pl.ds(c * B, B)])
    return _

  lax.fori_loop(0, pl.cdiv(NZC, NS), _out_chunk, None)


@functools.cache
def _make_sc_edge():
  return pl.kernel(
      _sc_body,
      out_type=jax.ShapeDtypeStruct((NC, N, ROW), jnp.float32),
      mesh=plsc.VectorSubcoreMesh(
          core_axis_name="c", subcore_axis_name="s",
          num_cores=NC, num_subcores=NS),
      scratch_types=[
          pltpu.VMEM_SHARED((N, ROW), jnp.float32),
          pltpu.VMEM((IBLK1, B), jnp.int32),
          pltpu.VMEM((IBLK1, B), jnp.int32),
          pltpu.VMEM((2, B, ROW), jnp.float32),
          pltpu.VMEM((2, B, 16), jnp.float32),
          pltpu.VMEM((2, B2, ROW), jnp.float32),
          pltpu.VMEM((B2,), jnp.int32),
          pltpu.SemaphoreType.DMA((2,)),
          pltpu.SemaphoreType.DMA((2,)),
          pltpu.SemaphoreType.DMA((2,)),
      ],
      compiler_params=pltpu.CompilerParams(use_tc_tiling_on_sc=False),
  )


def _fin_body(p0_ref, p1_ref, b_ref, out_ref):
  a0 = p0_ref[...]
  a1 = p1_ref[...]
  s = a0[:, :D] + a1[:, :D]
  d8 = a0[:, D:D + H] + a1[:, D:D + H]
  hh = lax.broadcasted_iota(jnp.int32, (H, D), 0)
  jj = lax.broadcasted_iota(jnp.int32, (H, D), 1)
  expand = (jj // OUT == hh).astype(jnp.float32)
  drep = jnp.dot(d8, expand, preferred_element_type=jnp.float32)
  x = s / jnp.maximum(drep, jnp.float32(1e-38)) + b_ref[...]
  out_ref[...] = jnp.where(x > 0.0, x, jnp.exp(x) - 1.0)


_fin = pl.pallas_call(
    _fin_body,
    grid=(N // _PREP_BLK,),
    in_specs=[
        pl.BlockSpec((_PREP_BLK, ROW), lambda i: (i, 0)),
        pl.BlockSpec((_PREP_BLK, ROW), lambda i: (i, 0)),
        pl.BlockSpec((1, D), lambda i: (0, 0)),
    ],
    out_specs=pl.BlockSpec((_PREP_BLK, D), lambda i: (i, 0)),
    out_shape=jax.ShapeDtypeStruct((N, D), jnp.float32),
)


def kernel(h, edge_index, W, attn_l, attn_r, bias):
  src = edge_index[0].astype(jnp.int32).reshape(E // B, B)
  dst = edge_index[1].astype(jnp.int32).reshape(E // B, B)

  mask = (jnp.arange(D)[:, None] // OUT == jnp.arange(H)[None, :])
  p_l = jnp.where(mask, attn_l.reshape(D, 1), 0.0)
  p_r = jnp.where(mask, attn_r.reshape(D, 1), 0.0)

  featel, er16 = _prep(h, W, p_l, p_r)
  partials = _make_sc_edge()(featel, er16, src, dst)
  out = _fin(partials[0], partials[1], bias.reshape(1, D))
  return out
